# baseline (speedup 1.0000x reference)
import jax
import jax.numpy as jnp
from jax.scipy.linalg import block_diag
from jax.experimental import pallas as pl

B, S, C, D = 1024, 200, 32, 32
NIH, NAH, NUH = 2, 2, 4

GS = 16
HR = B * S // 4 // GS
CR = B * C // 4 // GS
UR = B // 4 // GS
BBm = B // GS


def _fused_kernel(u0_ref, uemb_ref, h0_ref, hp_ref, ha_ref, hact_ref, hprod_ref,
                  c0_ref, cp_ref, ca_ref, cprod_ref,
                  w1_ref, w2p_ref, w2a_ref, w2s_ref,
                  w3p_ref, w3a_ref, w3act_ref, w3s_ref,
                  cand_out, cand_mask, hist_out, hist_mask, user_out, user_mask):
    f32 = jnp.float32
    user_out[...] = jnp.dot(uemb_ref[...], w1_ref[...], preferred_element_type=f32)
    user_mask[...] = u0_ref[...] != 0

    acc_c = jnp.dot(cp_ref[...], w2p_ref[...], preferred_element_type=f32)
    acc_c += jnp.dot(ca_ref[...], w2a_ref[...], preferred_element_type=f32)
    acc_c += jnp.dot(cprod_ref[...], w2s_ref[...], preferred_element_type=f32)
    cand_out[...] = acc_c
    cand_mask[...] = c0_ref[...] != 0

    acc_h = jnp.dot(hp_ref[...], w3p_ref[...], preferred_element_type=f32)
    acc_h += jnp.dot(ha_ref[...], w3a_ref[...], preferred_element_type=f32)
    acc_h += jnp.dot(hact_ref[...], w3act_ref[...], preferred_element_type=f32)
    acc_h += jnp.dot(hprod_ref[...], w3s_ref[...], preferred_element_type=f32)
    hist_out[...] = acc_h
    hist_mask[...] = h0_ref[...] != 0


def _bd4(w):
    return block_diag(w, w, w, w)


def kernel(user_hashes, user_embeddings, history_post_hashes, history_post_embeddings,
           history_author_embeddings, history_product_surface_embeddings,
           history_actions_embeddings, candidate_post_hashes, candidate_post_embeddings,
           candidate_author_embeddings, candidate_product_surface_embeddings,
           proj_mat_1, proj_mat_2, proj_mat_3):
    u0 = user_hashes[:, :1].astype(jnp.int32)
    h0 = history_post_hashes[:, :, 0].astype(jnp.int32)
    c0 = candidate_post_hashes[:, :, 0].astype(jnp.int32)
    uemb = user_embeddings.reshape(B // 4, 4 * NUH * D)
    hp = history_post_embeddings.reshape(B * S // 4, 4 * NIH * D)
    ha = history_author_embeddings.reshape(B * S // 4, 4 * NAH * D)
    hact = history_actions_embeddings.reshape(B * S // 4, 4 * D)
    hprod = history_product_surface_embeddings.reshape(B * S // 4, 4 * D)
    cp = candidate_post_embeddings.reshape(B * C // 4, 4 * NIH * D)
    ca = candidate_author_embeddings.reshape(B * C // 4, 4 * NAH * D)
    cprod = candidate_product_surface_embeddings.reshape(B * C // 4, 4 * D)

    w1b = _bd4(proj_mat_1)
    w2p = _bd4(proj_mat_2[: NIH * D])
    w2a = _bd4(proj_mat_2[NIH * D:(NIH + NAH) * D])
    w2s = _bd4(proj_mat_2[(NIH + NAH) * D:])
    w3p = _bd4(proj_mat_3[: NIH * D])
    w3a = _bd4(proj_mat_3[NIH * D:(NIH + NAH) * D])
    w3act = _bd4(proj_mat_3[(NIH + NAH) * D:(NIH + NAH + 1) * D])
    w3s = _bd4(proj_mat_3[(NIH + NAH + 1) * D:])

    grid = (GS,)
    bspec = lambda r, c: pl.BlockSpec((r, c), lambda i: (i, 0))
    wspec = lambda r, c: pl.BlockSpec((r, c), lambda i: (0, 0))

    out_shapes = (
        jax.ShapeDtypeStruct((B * C // 4, 128), jnp.float32),
        jax.ShapeDtypeStruct((B, C), jnp.bool_),
        jax.ShapeDtypeStruct((B * S // 4, 128), jnp.float32),
        jax.ShapeDtypeStruct((B, S), jnp.bool_),
        jax.ShapeDtypeStruct((B // 4, 128), jnp.float32),
        jax.ShapeDtypeStruct((B, 1), jnp.bool_),
    )
    out_specs = (
        bspec(CR, 128),
        bspec(BBm, C),
        bspec(HR, 128),
        bspec(BBm, S),
        bspec(UR, 128),
        bspec(BBm, 1),
    )
    in_specs = (
        bspec(BBm, 1),
        bspec(UR, 512),
        bspec(BBm, S),
        bspec(HR, 256),
        bspec(HR, 256),
        bspec(HR, 128),
        bspec(HR, 128),
        bspec(BBm, C),
        bspec(CR, 256),
        bspec(CR, 256),
        bspec(CR, 128),
        wspec(4 * NUH * D, 128),
        wspec(256, 128), wspec(256, 128), wspec(128, 128),
        wspec(256, 128), wspec(256, 128), wspec(128, 128), wspec(128, 128),
    )

    cand3, cand_mask, hist3, hist_mask, user3, user_mask = pl.pallas_call(
        _fused_kernel,
        grid=grid,
        in_specs=in_specs,
        out_specs=out_specs,
        out_shape=out_shapes,
    )(u0, uemb, h0, hp, ha, hact, hprod, c0, cp, ca, cprod,
      w1b, w2p, w2a, w2s, w3p, w3a, w3act, w3s)

    return (cand3.reshape(B, C, D), cand_mask,
            hist3.reshape(B, S, D), hist_mask,
            user3.reshape(B, 1, D), user_mask)


# speedup vs baseline: 19.3919x; 19.3919x over previous
import jax
import jax.numpy as jnp
from jax.scipy.linalg import block_diag
from jax.experimental import pallas as pl

B, S, C, D = 1024, 200, 32, 32
NIH, NAH, NUH = 2, 2, 4

SCH = 8
CCH = 8


def _make_seq_kernel(nparts):
    def body(*refs):
        xs, bds, out_ref = refs[:nparts], refs[nparts:2 * nparts], refs[-1]
        f32 = jnp.float32
        for j in range(out_ref.shape[0] // 128):
            acc = None
            for x_ref, bd_ref in zip(xs, bds):
                k = bd_ref.shape[1]
                p = jnp.dot(bd_ref[...], x_ref[j * k:(j + 1) * k, :],
                            preferred_element_type=f32)
                acc = p if acc is None else acc + p
            out_ref[j * 128:(j + 1) * 128, :] = acc
    return body


def _user_mask_kernel(xu_ref, wt1_ref, u0_ref, h0_ref, c0_ref,
                      uout_ref, umask_ref, hmask_ref, cmask_ref):
    uout_ref[...] = jnp.dot(wt1_ref[...], xu_ref[...],
                            preferred_element_type=jnp.float32)
    umask_ref[...] = u0_ref[0:1, :] != 0
    hmask_ref[...] = h0_ref[:, 0, :] != 0
    cmask_ref[...] = c0_ref[:, 0, :] != 0


def _bd4(w):
    return block_diag(w, w, w, w)


def kernel(user_hashes, user_embeddings, history_post_hashes, history_post_embeddings,
           history_author_embeddings, history_product_surface_embeddings,
           history_actions_embeddings, candidate_post_hashes, candidate_post_embeddings,
           candidate_author_embeddings, candidate_product_surface_embeddings,
           proj_mat_1, proj_mat_2, proj_mat_3):
    f32 = jnp.float32
    xp = history_post_embeddings.transpose(1, 2, 3, 0).reshape(S * NIH * D, B)
    xa = history_author_embeddings.transpose(1, 2, 3, 0).reshape(S * NAH * D, B)
    xact = history_actions_embeddings.transpose(1, 2, 0).reshape(S * D, B)
    xprod = history_product_surface_embeddings.transpose(1, 2, 0).reshape(S * D, B)
    xcp = candidate_post_embeddings.transpose(1, 2, 3, 0).reshape(C * NIH * D, B)
    xca = candidate_author_embeddings.transpose(1, 2, 3, 0).reshape(C * NAH * D, B)
    xcs = candidate_product_surface_embeddings.transpose(1, 2, 0).reshape(C * D, B)
    xu = user_embeddings.transpose(1, 2, 0).reshape(NUH * D, B)
    u0 = user_hashes.transpose(1, 0).astype(jnp.int32)
    h0 = history_post_hashes.transpose(1, 2, 0).astype(jnp.int32)
    c0 = candidate_post_hashes.transpose(1, 2, 0).astype(jnp.int32)

    wt1 = proj_mat_1.T
    wt2, wt3 = proj_mat_2.T, proj_mat_3.T
    bd2p = _bd4(wt2[:, : NIH * D])
    bd2a = _bd4(wt2[:, NIH * D:(NIH + NAH) * D])
    bd2s = _bd4(wt2[:, (NIH + NAH) * D:])
    bd3p = _bd4(wt3[:, : NIH * D])
    bd3a = _bd4(wt3[:, NIH * D:(NIH + NAH) * D])
    bd3act = _bd4(wt3[:, (NIH + NAH) * D:(NIH + NAH + 1) * D])
    bd3s = _bd4(wt3[:, (NIH + NAH + 1) * D:])

    bspec = lambda r, c: pl.BlockSpec((r, c), lambda i: (i, 0))
    wspec = lambda r, c: pl.BlockSpec((r, c), lambda i: (0, 0))

    hist = pl.pallas_call(
        _make_seq_kernel(4),
        grid=(S // SCH,),
        in_specs=(
            bspec(SCH * NIH * D, B), bspec(SCH * NAH * D, B),
            bspec(SCH * D, B), bspec(SCH * D, B),
            wspec(128, 256), wspec(128, 256), wspec(128, 128), wspec(128, 128),
        ),
        out_specs=bspec(SCH * D, B),
        out_shape=jax.ShapeDtypeStruct((S * D, B), f32),
    )(xp, xa, xact, xprod, bd3p, bd3a, bd3act, bd3s)

    cand = pl.pallas_call(
        _make_seq_kernel(3),
        grid=(C // CCH,),
        in_specs=(
            bspec(CCH * NIH * D, B), bspec(CCH * NAH * D, B),
            bspec(CCH * D, B),
            wspec(128, 256), wspec(128, 256), wspec(128, 128),
        ),
        out_specs=bspec(CCH * D, B),
        out_shape=jax.ShapeDtypeStruct((C * D, B), f32),
    )(xcp, xca, xcs, bd2p, bd2a, bd2s)

    user, umask, hmask, cmask = pl.pallas_call(
        _user_mask_kernel,
        grid=(1,),
        in_specs=(
            wspec(NUH * D, B), wspec(D, NUH * D),
            wspec(NUH, B),
            pl.BlockSpec((S, NIH, B), lambda i: (0, 0, 0)),
            pl.BlockSpec((C, NIH, B), lambda i: (0, 0, 0)),
        ),
        out_specs=(
            wspec(D, B), wspec(1, B), wspec(S, B), wspec(C, B),
        ),
        out_shape=(
            jax.ShapeDtypeStruct((D, B), f32),
            jax.ShapeDtypeStruct((1, B), jnp.bool_),
            jax.ShapeDtypeStruct((S, B), jnp.bool_),
            jax.ShapeDtypeStruct((C, B), jnp.bool_),
        ),
    )(xu, wt1, u0, h0, c0)

    return (cand.reshape(C, D, B).transpose(2, 0, 1),
            cmask.transpose(1, 0),
            hist.reshape(S, D, B).transpose(2, 0, 1),
            hmask.transpose(1, 0),
            user.transpose(1, 0).reshape(B, 1, D),
            umask.transpose(1, 0))


# speedup vs baseline: 20.0182x; 1.0323x over previous
import jax
import jax.numpy as jnp
from jax.scipy.linalg import block_diag
from jax.experimental import pallas as pl

B, S, C, D = 1024, 200, 32, 32
NIH, NAH, NUH = 2, 2, 4

SCH = 20
CCH = 16


def _make_seq_kernel(nparts):
    def body(*refs):
        xs, bds, out_ref = refs[:nparts], refs[nparts:2 * nparts], refs[-1]
        f32 = jnp.float32
        for j in range(out_ref.shape[0] // 128):
            acc = None
            for x_ref, bd_ref in zip(xs, bds):
                k = bd_ref.shape[1]
                p = jnp.dot(bd_ref[...], x_ref[j * k:(j + 1) * k, :],
                            preferred_element_type=f32)
                acc = p if acc is None else acc + p
            out_ref[j * 128:(j + 1) * 128, :] = acc
    return body


def _user_mask_kernel(xu_ref, wt1_ref, u0_ref, h0_ref, c0_ref,
                      uout_ref, umask_ref, hmask_ref, cmask_ref):
    uout_ref[...] = jnp.dot(wt1_ref[...], xu_ref[...],
                            preferred_element_type=jnp.float32)
    umask_ref[...] = u0_ref[0:1, :] != 0
    hmask_ref[...] = h0_ref[:, 0, :] != 0
    cmask_ref[...] = c0_ref[:, 0, :] != 0


def _bd4(w):
    return block_diag(w, w, w, w)


def kernel(user_hashes, user_embeddings, history_post_hashes, history_post_embeddings,
           history_author_embeddings, history_product_surface_embeddings,
           history_actions_embeddings, candidate_post_hashes, candidate_post_embeddings,
           candidate_author_embeddings, candidate_product_surface_embeddings,
           proj_mat_1, proj_mat_2, proj_mat_3):
    f32 = jnp.float32
    xp = history_post_embeddings.transpose(1, 2, 3, 0).reshape(S * NIH * D, B)
    xa = history_author_embeddings.transpose(1, 2, 3, 0).reshape(S * NAH * D, B)
    xact = history_actions_embeddings.transpose(1, 2, 0).reshape(S * D, B)
    xprod = history_product_surface_embeddings.transpose(1, 2, 0).reshape(S * D, B)
    xcp = candidate_post_embeddings.transpose(1, 2, 3, 0).reshape(C * NIH * D, B)
    xca = candidate_author_embeddings.transpose(1, 2, 3, 0).reshape(C * NAH * D, B)
    xcs = candidate_product_surface_embeddings.transpose(1, 2, 0).reshape(C * D, B)
    xu = user_embeddings.transpose(1, 2, 0).reshape(NUH * D, B)
    u0 = user_hashes.transpose(1, 0).astype(jnp.int32)
    h0 = history_post_hashes.transpose(1, 2, 0).astype(jnp.int32)
    c0 = candidate_post_hashes.transpose(1, 2, 0).astype(jnp.int32)

    wt1 = proj_mat_1.T
    wt2, wt3 = proj_mat_2.T, proj_mat_3.T
    bd2p = _bd4(wt2[:, : NIH * D])
    bd2a = _bd4(wt2[:, NIH * D:(NIH + NAH) * D])
    bd2s = _bd4(wt2[:, (NIH + NAH) * D:])
    bd3p = _bd4(wt3[:, : NIH * D])
    bd3a = _bd4(wt3[:, NIH * D:(NIH + NAH) * D])
    bd3act = _bd4(wt3[:, (NIH + NAH) * D:(NIH + NAH + 1) * D])
    bd3s = _bd4(wt3[:, (NIH + NAH + 1) * D:])

    bspec = lambda r, c: pl.BlockSpec((r, c), lambda i: (i, 0))
    wspec = lambda r, c: pl.BlockSpec((r, c), lambda i: (0, 0))

    hist = pl.pallas_call(
        _make_seq_kernel(4),
        grid=(S // SCH,),
        in_specs=(
            bspec(SCH * NIH * D, B), bspec(SCH * NAH * D, B),
            bspec(SCH * D, B), bspec(SCH * D, B),
            wspec(128, 256), wspec(128, 256), wspec(128, 128), wspec(128, 128),
        ),
        out_specs=bspec(SCH * D, B),
        out_shape=jax.ShapeDtypeStruct((S * D, B), f32),
    )(xp, xa, xact, xprod, bd3p, bd3a, bd3act, bd3s)

    cand = pl.pallas_call(
        _make_seq_kernel(3),
        grid=(C // CCH,),
        in_specs=(
            bspec(CCH * NIH * D, B), bspec(CCH * NAH * D, B),
            bspec(CCH * D, B),
            wspec(128, 256), wspec(128, 256), wspec(128, 128),
        ),
        out_specs=bspec(CCH * D, B),
        out_shape=jax.ShapeDtypeStruct((C * D, B), f32),
    )(xcp, xca, xcs, bd2p, bd2a, bd2s)

    user, umask, hmask, cmask = pl.pallas_call(
        _user_mask_kernel,
        grid=(1,),
        in_specs=(
            wspec(NUH * D, B), wspec(D, NUH * D),
            wspec(NUH, B),
            pl.BlockSpec((S, NIH, B), lambda i: (0, 0, 0)),
            pl.BlockSpec((C, NIH, B), lambda i: (0, 0, 0)),
        ),
        out_specs=(
            wspec(D, B), wspec(1, B), wspec(S, B), wspec(C, B),
        ),
        out_shape=(
            jax.ShapeDtypeStruct((D, B), f32),
            jax.ShapeDtypeStruct((1, B), jnp.bool_),
            jax.ShapeDtypeStruct((S, B), jnp.bool_),
            jax.ShapeDtypeStruct((C, B), jnp.bool_),
        ),
    )(xu, wt1, u0, h0, c0)

    return (cand.reshape(C, D, B).transpose(2, 0, 1),
            cmask.transpose(1, 0),
            hist.reshape(S, D, B).transpose(2, 0, 1),
            hmask.transpose(1, 0),
            user.transpose(1, 0).reshape(B, 1, D),
            umask.transpose(1, 0))
